# Initial kernel scaffold; baseline (speedup 1.0000x reference)
import jax
import jax.numpy as jnp
from jax.experimental import pallas as pl


def kernel(input, W, b):
    raise NotImplementedError("write your pallas kernel here")


# speedup vs baseline: 2.0645x; 2.0645x over previous
import jax
import jax.numpy as jnp
from jax import lax
from jax.experimental import pallas as pl
from jax.experimental.pallas import tpu as pltpu

_BATCH = 4096
_FEAT = 8192
_RB = 256
_BB = 256
_CB = 2048

_KS0 = 0
_KS1 = 42
_KS2 = (0x1BD11BDA ^ _KS0 ^ _KS1) & 0xFFFFFFFF
_ROT_A = (13, 15, 26, 6)
_ROT_B = (17, 29, 16, 24)


def _c(v):
    return jnp.int32(v if v < 2**31 else v - 2**32)


def _rotl(x, r):
    return lax.shift_left(x, _c(r)) | lax.shift_right_logical(x, _c(32 - r))


def _mix(x0, x1, rots):
    for r in rots:
        x0 = x0 + x1
        x1 = _rotl(x1, r)
        x1 = x1 ^ x0
    return x0, x1


def _keep_mask(x1):
    x0 = x1
    x1 = _rotl(x1, _ROT_A[0]) ^ x0
    x0, x1 = _mix(x0, x1, _ROT_A[1:])
    x0, x1 = x0 + _c(_KS1), x1 + _c(_KS2 + 1)
    x0, x1 = _mix(x0, x1, _ROT_B)
    x0, x1 = x0 + _c(_KS2), x1 + _c(_KS0 + 2)
    x0, x1 = _mix(x0, x1, _ROT_A)
    x0, x1 = x0 + _c(_KS0), x1 + _c(_KS1 + 3)
    x0, x1 = _mix(x0, x1, _ROT_B)
    x0, x1 = x0 + _c(_KS1), x1 + _c(_KS2 + 4)
    x0, x1 = _mix(x0, x1, _ROT_A)
    x0, x1 = x0 + _c(_KS2), x1 + _c(_KS0 + 5)
    bits = x0 ^ x1
    return bits >= 0


def _rank_body(s_row_ref, s_col_ref, rank_ref):
    i = pl.program_id(0)
    s_all = s_row_ref[...]
    s_col = s_col_ref[...]
    gt = (s_all > s_col).astype(jnp.int32)
    idx_all = lax.broadcasted_iota(jnp.int32, (_RB, _BATCH), 1)
    row_ids = i * _RB + lax.broadcasted_iota(jnp.int32, (_RB, 1), 0)
    tie = ((s_all == s_col) & (idx_all < row_ids)).astype(jnp.int32)
    rank_ref[...] = jnp.sum(gt + tie, axis=1, keepdims=True)


def _apply_body(rank_ref, x_ref, o_ref):
    j = pl.program_id(1)
    r = rank_ref[...]
    base = lax.shift_left(r, _c(13)) + (j * _c(_CB) + _c(_KS1))
    k = lax.broadcasted_iota(jnp.int32, (_BB, _CB), 1)
    x1 = base + k
    keep = _keep_mask(x1)
    x = x_ref[...]
    o_ref[...] = jnp.where(keep, x + x, jnp.zeros_like(x))


def kernel(input, W, b):
    gap = jnp.mean(input, axis=(2, 3))
    logitsT = jnp.einsum("ok,bk->ob", W, gap) + b[:, None]
    h_xT = jax.nn.softmax(logitsT, axis=0)
    score = h_xT[0, :]

    s_row = score.reshape(1, _BATCH)
    s_col = score.reshape(_BATCH, 1)
    rank = pl.pallas_call(
        _rank_body,
        grid=(_BATCH // _RB,),
        in_specs=[
            pl.BlockSpec((1, _BATCH), lambda i: (0, 0)),
            pl.BlockSpec((_RB, 1), lambda i: (i, 0)),
        ],
        out_specs=pl.BlockSpec((_RB, 1), lambda i: (i, 0)),
        out_shape=jax.ShapeDtypeStruct((_BATCH, 1), jnp.int32),
        compiler_params=pltpu.CompilerParams(
            dimension_semantics=("arbitrary",)),
    )(s_row, s_col)

    x2d = input.reshape(_BATCH, _FEAT)
    out2d = pl.pallas_call(
        _apply_body,
        grid=(_BATCH // _BB, _FEAT // _CB),
        in_specs=[
            pl.BlockSpec((_BB, 1), lambda i, j: (i, 0)),
            pl.BlockSpec((_BB, _CB), lambda i, j: (i, j)),
        ],
        out_specs=pl.BlockSpec((_BB, _CB), lambda i, j: (i, j)),
        out_shape=jax.ShapeDtypeStruct((_BATCH, _FEAT), jnp.float32),
        compiler_params=pltpu.CompilerParams(
            dimension_semantics=("parallel", "parallel")),
    )(rank, x2d)
    return out2d.reshape(input.shape)
